# baseline (speedup 1.0000x reference)
import functools
import jax
import jax.numpy as jnp
from jax import lax
from jax.experimental import pallas as pl
from jax.experimental.pallas import tpu as pltpu
from jax.experimental.pallas import tpu_sc as plsc

_NUM_BUCKETS = 12
_EMB_DIM = 32
_BOUNDS = (1.0, 2.0, 5.0, 10.0, 20.0, 50.0, 100.0, 200.0, 500.0, 1000.0, 2000.0)

_NC = 2
_NS = 16
_NW = _NC * _NS
_L = 16
_GI = 128


@functools.lru_cache(maxsize=None)
def _build_sc_call(n):
    per_w = n // _NW
    c = 2560
    while per_w % c:
        c //= 2
    iters = per_w // c
    groups = c // _L
    n_gather = c // _GI

    @functools.partial(
        pl.kernel,
        mesh=plsc.VectorSubcoreMesh(core_axis_name="c", subcore_axis_name="s"),
        out_type=jax.ShapeDtypeStruct((n, _EMB_DIM), jnp.float32),
        scratch_types=[
            pltpu.VMEM((c,), jnp.float32),
            pltpu.VMEM((c,), jnp.int32),
            pltpu.VMEM((c, _EMB_DIM), jnp.float32),
            pltpu.SemaphoreType.DMA,
        ],
        compiler_params=pltpu.CompilerParams(
            needs_layout_passes=False, use_tc_tiling_on_sc=False
        ),
    )
    def sc_call(amounts_hbm, emb_hbm, out_hbm, amt_v, idx_v, rows_v, sem):
        wid = lax.axis_index("s") * _NC + lax.axis_index("c")
        base = wid * per_w

        def chunk(i, carry):
            off = base + i * c
            pltpu.sync_copy(amounts_hbm.at[pl.ds(off, c)], amt_v)

            def group(g, carry2):
                a = amt_v[pl.ds(g * _L, _L)]
                acc = jnp.zeros((_L,), jnp.int32)
                for b in _BOUNDS:
                    acc = acc + jnp.where(a >= b, 1, 0)
                idx_v[pl.ds(g * _L, _L)] = acc
                return carry2

            lax.fori_loop(0, groups, group, 0)

            copies = [
                pltpu.async_copy(
                    emb_hbm.at[idx_v.at[pl.ds(j * _GI, _GI)]],
                    rows_v.at[pl.ds(j * _GI, _GI)],
                    sem,
                )
                for j in range(n_gather)
            ]
            for cp in copies:
                cp.wait()
            pltpu.sync_copy(rows_v, out_hbm.at[pl.ds(off, c)])
            return carry

        lax.fori_loop(0, iters, chunk, 0)

    return sc_call


def kernel(amounts, emb):
    bsz, seq = amounts.shape
    n = bsz * seq
    out = _build_sc_call(n)(amounts.reshape(n), emb)
    return out.reshape(bsz, seq, _EMB_DIM)


# speedup vs baseline: 6.7228x; 6.7228x over previous
import functools
import jax
import jax.numpy as jnp
from jax import lax
from jax.experimental import pallas as pl
from jax.experimental.pallas import tpu as pltpu
from jax.experimental.pallas import tpu_sc as plsc

_NUM_BUCKETS = 12
_EMB_DIM = 32
_BOUNDS = (1.0, 2.0, 5.0, 10.0, 20.0, 50.0, 100.0, 200.0, 500.0, 1000.0, 2000.0)

_NC = 2
_NS = 16
_NW = _NC * _NS
_L = 16


@functools.lru_cache(maxsize=None)
def _build_sc_call(n):
    per_w = n // _NW
    c = 3200
    while per_w % c:
        c //= 2
    iters = per_w // c
    groups = c // _L

    @functools.partial(
        pl.kernel,
        mesh=plsc.VectorSubcoreMesh(core_axis_name="c", subcore_axis_name="s"),
        out_type=jax.ShapeDtypeStruct((n * _EMB_DIM,), jnp.float32),
        scratch_types=[
            pltpu.VMEM((_NUM_BUCKETS * _EMB_DIM,), jnp.float32),
            pltpu.VMEM((c,), jnp.float32),
            pltpu.VMEM((c * _EMB_DIM,), jnp.float32),
        ],
        compiler_params=pltpu.CompilerParams(needs_layout_passes=False),
    )
    def sc_call(amounts_hbm, emb_hbm, out_hbm, emb_v, amt_v, out_v):
        wid = lax.axis_index("s") * _NC + lax.axis_index("c")
        base = wid * per_w
        pltpu.sync_copy(emb_hbm, emb_v)
        jbase = lax.iota(jnp.int32, _L) * _EMB_DIM

        def chunk(i, carry):
            off = base + i * c
            pltpu.sync_copy(amounts_hbm.at[pl.ds(off, c)], amt_v)

            @plsc.parallel_loop(0, groups, unroll=4)
            def group(g):
                a = amt_v[pl.ds(g * _L, _L)]
                acc = jnp.zeros((_L,), jnp.int32)
                for b in _BOUNDS:
                    acc = acc + jnp.where(a >= b, _EMB_DIM, 0)
                svec = jbase + g * (_L * _EMB_DIM)
                for k in range(_EMB_DIM):
                    vals = plsc.load_gather(emb_v, [acc + k])
                    plsc.store_scatter(out_v, [svec + k], vals)

            pltpu.sync_copy(out_v, out_hbm.at[pl.ds(off * _EMB_DIM, c * _EMB_DIM)])
            return carry

        lax.fori_loop(0, iters, chunk, 0)

    return sc_call


def kernel(amounts, emb):
    bsz, seq = amounts.shape
    n = bsz * seq
    out = _build_sc_call(n)(
        amounts.reshape(n), emb.reshape(_NUM_BUCKETS * _EMB_DIM)
    )
    return out.reshape(bsz, seq, _EMB_DIM)


# speedup vs baseline: 24.6567x; 3.6676x over previous
import functools
import jax
import jax.numpy as jnp
from jax import lax
from jax.experimental import pallas as pl
from jax.experimental.pallas import tpu as pltpu
from jax.experimental.pallas import tpu_sc as plsc

_NUM_BUCKETS = 12
_EMB_DIM = 32
_BOUNDS = (1.0, 2.0, 5.0, 10.0, 20.0, 50.0, 100.0, 200.0, 500.0, 1000.0, 2000.0)

_NC = 2
_NS = 16
_NW = _NC * _NS
_L = 16
_H = 800


@functools.lru_cache(maxsize=None)
def _build_sc_call(bsz, seq):
    rows_per_w = bsz // _NW
    cr = 8
    c = cr * seq
    halves = c // _H
    iters = rows_per_w // cr
    groups = _H // _L

    @functools.partial(
        pl.kernel,
        mesh=plsc.VectorSubcoreMesh(core_axis_name="c", subcore_axis_name="s"),
        out_type=jax.ShapeDtypeStruct((bsz * seq // 8, 8, _EMB_DIM), jnp.float32),
        scratch_types=[
            pltpu.VMEM((_NUM_BUCKETS, _EMB_DIM), jnp.float32),
            pltpu.VMEM((_NUM_BUCKETS * _EMB_DIM,), jnp.float32),
            pltpu.VMEM((cr, seq), jnp.float32),
            pltpu.VMEM((_H // 8, 8, _EMB_DIM), jnp.float32),
        ],
        compiler_params=pltpu.CompilerParams(needs_layout_passes=False),
    )
    def sc_call(amounts_hbm, emb_hbm, out_hbm, emb2, embf, amt_v, rows3):
        wid = lax.axis_index("s") * _NC + lax.axis_index("c")
        row_base = wid * rows_per_w
        pltpu.sync_copy(emb_hbm, emb2)
        for r in range(_NUM_BUCKETS):
            embf[pl.ds(r * _EMB_DIM, _L)] = emb2[r, pl.ds(0, _L)]
            embf[pl.ds(r * _EMB_DIM + _L, _L)] = emb2[r, pl.ds(_L, _L)]
        lane = lax.iota(jnp.int32, _L)

        def chunk(i, carry):
            r0 = row_base + i * cr
            pltpu.sync_copy(amounts_hbm.at[pl.ds(r0, cr), :], amt_v)
            for h in range(halves):

                @plsc.parallel_loop(0, groups, unroll=2)
                def group(g):
                    tvec = h * _H + g * _L + lane
                    rv = (tvec * 5243) >> 20
                    tv = tvec - rv * seq
                    a = plsc.load_gather(amt_v, [rv, tv])
                    acc = jnp.zeros((_L,), jnp.int32)
                    for b in _BOUNDS:
                        acc = acc + jnp.where(a >= b, _EMB_DIM, 0)
                    g2 = g * 2
                    for j in range(_L):
                        off = acc[j]
                        lo = embf[pl.ds(off, _L)]
                        hi = embf[pl.ds(off + _L, _L)]
                        blk = g2 + (j >> 3)
                        sub = j & 7
                        rows3[blk, sub, pl.ds(0, _L)] = lo
                        rows3[blk, sub, pl.ds(_L, _L)] = hi

                q0 = r0 * (seq // 8) + h * (_H // 8)
                pltpu.sync_copy(rows3, out_hbm.at[pl.ds(q0, _H // 8), :, :])
            return carry

        lax.fori_loop(0, iters, chunk, 0)

    return sc_call


def kernel(amounts, emb):
    bsz, seq = amounts.shape
    out = _build_sc_call(bsz, seq)(amounts, emb)
    return out.reshape(bsz, seq, _EMB_DIM)


# speedup vs baseline: 127.1279x; 5.1559x over previous
import functools
import jax
import jax.numpy as jnp
from jax import lax
from jax.experimental import pallas as pl
from jax.experimental.pallas import tpu as pltpu
from jax.experimental.pallas import tpu_sc as plsc

_NUM_BUCKETS = 12
_EMB_DIM = 32
_BOUNDS = (1.0, 2.0, 5.0, 10.0, 20.0, 50.0, 100.0, 200.0, 500.0, 1000.0, 2000.0)

_NC = 2
_NS = 16
_NW = _NC * _NS
_L = 16
_BT = 128
_TS = 10


@functools.lru_cache(maxsize=None)
def _build_sc_call(bsz, seq):
    btiles_per_w = bsz // (_NW * _BT)
    spans = seq // _TS
    kt = _EMB_DIM // 8

    @functools.partial(
        pl.kernel,
        mesh=plsc.VectorSubcoreMesh(core_axis_name="c", subcore_axis_name="s"),
        out_type=jax.ShapeDtypeStruct(
            (seq, kt, bsz // _BT, 8, _BT), jnp.float32
        ),
        scratch_types=[
            pltpu.VMEM((_NUM_BUCKETS, _EMB_DIM), jnp.float32),
            pltpu.VMEM((_BT, seq), jnp.float32),
            pltpu.VMEM((2 * _TS, kt, 1, 8, _BT), jnp.float32),
            pltpu.SemaphoreType.DMA,
        ],
        compiler_params=pltpu.CompilerParams(needs_layout_passes=False),
    )
    def sc_call(amounts_hbm, emb_hbm, out_hbm, emb2, amt_v, blk_v, sem):
        wid = lax.axis_index("s") * _NC + lax.axis_index("c")
        bt_base = wid * btiles_per_w
        pltpu.sync_copy(emb_hbm, emb2)
        lane = lax.iota(jnp.int32, _L)
        rclamp = jnp.minimum(lane, _NUM_BUCKETS - 1)
        cols = [
            plsc.load_gather(emb2, [rclamp, jnp.full((_L,), k, jnp.int32)])
            for k in range(_EMB_DIM)
        ]

        def out_slice(t0):
            return lambda bt: out_hbm.at[pl.ds(t0, _TS), :, pl.ds(bt, 1), :, :]

        def btile(i, carry):
            bt = bt_base + i
            pltpu.sync_copy(amounts_hbm.at[pl.ds(bt * _BT, _BT), :], amt_v)

            def span(h, carry2):
                t0 = h * _TS
                base = (h % 2) * _TS

                @pl.when(h >= 2)
                def _wait_prev():
                    pltpu.make_async_copy(
                        blk_v.at[pl.ds(base, _TS)],
                        out_hbm.at[pl.ds(t0 - 2 * _TS, _TS), :, pl.ds(bt, 1), :, :],
                        sem,
                    ).wait()

                @plsc.parallel_loop(0, _TS)
                def trow(tl):
                    t = t0 + tl
                    row = base + tl
                    for bg in range(_BT // _L):
                        bvec = bg * _L + lane
                        a = plsc.load_gather(
                            amt_v, [bvec, jnp.full((_L,), 0, jnp.int32) + t]
                        )
                        acc = jnp.zeros((_L,), jnp.int32)
                        for b in _BOUNDS:
                            acc = acc + jnp.where(a >= b, 1, 0)
                        for k in range(_EMB_DIM):
                            vals = cols[k].at[acc].get(
                                mode="promise_in_bounds"
                            )
                            blk_v[row, k // 8, 0, k % 8, pl.ds(bg * _L, _L)] = vals

                pltpu.async_copy(
                    blk_v.at[pl.ds(base, _TS)],
                    out_hbm.at[pl.ds(t0, _TS), :, pl.ds(bt, 1), :, :],
                    sem,
                )
                return carry2

            lax.fori_loop(0, spans, span, 0)
            for d in (spans - 2, spans - 1):
                pltpu.make_async_copy(
                    blk_v.at[pl.ds((d % 2) * _TS, _TS)],
                    out_hbm.at[pl.ds(d * _TS, _TS), :, pl.ds(bt, 1), :, :],
                    sem,
                ).wait()
            return carry

        lax.fori_loop(0, btiles_per_w, btile, 0)

    return sc_call


def kernel(amounts, emb):
    bsz, seq = amounts.shape
    out5 = _build_sc_call(bsz, seq)(amounts, emb)
    out = out5.transpose(2, 4, 0, 1, 3).reshape(bsz, seq, _EMB_DIM)
    return out


# speedup vs baseline: 129.5383x; 1.0190x over previous
import functools
import jax
import jax.numpy as jnp
from jax import lax
from jax.experimental import pallas as pl
from jax.experimental.pallas import tpu as pltpu
from jax.experimental.pallas import tpu_sc as plsc

_NUM_BUCKETS = 12
_EMB_DIM = 32
_BOUNDS = (1.0, 2.0, 5.0, 10.0, 20.0, 50.0, 100.0, 200.0, 500.0, 1000.0, 2000.0)

_NC = 2
_NS = 16
_NW = _NC * _NS
_L = 16
_BT = 128
_TS = 5


@functools.lru_cache(maxsize=None)
def _build_sc_call(bsz, seq):
    btiles_per_w = bsz // (_NW * _BT)
    spans = seq // _TS
    kt = _EMB_DIM // 8

    @functools.partial(
        pl.kernel,
        mesh=plsc.VectorSubcoreMesh(core_axis_name="c", subcore_axis_name="s"),
        out_type=jax.ShapeDtypeStruct(
            (seq, kt, bsz // _BT, 8, _BT), jnp.float32
        ),
        scratch_types=[
            pltpu.VMEM((_NUM_BUCKETS, _EMB_DIM), jnp.float32),
            pltpu.VMEM((2 * _BT, seq), jnp.float32),
            pltpu.VMEM((2 * _TS, kt, 1, 8, _BT), jnp.float32),
            pltpu.SemaphoreType.DMA,
            pltpu.SemaphoreType.DMA,
        ],
        compiler_params=pltpu.CompilerParams(needs_layout_passes=False),
    )
    def sc_call(amounts_hbm, emb_hbm, out_hbm, emb2, amt_v, blk_v, osem, asem):
        wid = lax.axis_index("s") * _NC + lax.axis_index("c")
        bt_base = wid * btiles_per_w
        pltpu.sync_copy(emb_hbm, emb2)
        lane = lax.iota(jnp.int32, _L)
        rclamp = jnp.minimum(lane, _NUM_BUCKETS - 1)
        cols = [
            plsc.load_gather(emb2, [rclamp, jnp.full((_L,), k, jnp.int32)])
            for k in range(_EMB_DIM)
        ]

        def amt_copy(i):
            return pltpu.make_async_copy(
                amounts_hbm.at[pl.ds((bt_base + i) * _BT, _BT), :],
                amt_v.at[pl.ds((i % 2) * _BT, _BT), :],
                asem,
            )

        def out_copy(i, h):
            return pltpu.make_async_copy(
                blk_v.at[pl.ds(((i + h) % 2) * _TS, _TS)],
                out_hbm.at[
                    pl.ds(h * _TS, _TS), :, pl.ds(bt_base + i, 1), :, :
                ],
                osem,
            )

        amt_copy(0).start()

        def btile(i, carry):
            def span(h, carry2):
                @pl.when(h == 0)
                def _wait_amt():
                    amt_copy(i).wait()

                @pl.when(jnp.logical_and(h == 1, i + 1 < btiles_per_w))
                def _prefetch_amt():
                    amt_copy(i + 1).start()

                @pl.when(h >= 2)
                def _wait_same_btile():
                    out_copy(i, h - 2).wait()

                @pl.when(jnp.logical_and(h < 2, i > 0))
                def _wait_prev_btile():
                    out_copy(i - 1, spans - 2 + h).wait()

                base = ((i + h) % 2) * _TS
                arow = (i % 2) * _BT
                t0 = h * _TS

                @plsc.parallel_loop(0, _TS)
                def trow(tl):
                    t = t0 + tl
                    row = base + tl
                    for bg in range(_BT // _L):
                        bvec = arow + bg * _L + lane
                        a = plsc.load_gather(
                            amt_v, [bvec, jnp.full((_L,), 0, jnp.int32) + t]
                        )
                        acc = jnp.zeros((_L,), jnp.int32)
                        for b in _BOUNDS:
                            acc = acc + jnp.where(a >= b, 1, 0)
                        for k in range(_EMB_DIM):
                            vals = cols[k].at[acc].get(
                                mode="promise_in_bounds"
                            )
                            blk_v[row, k // 8, 0, k % 8, pl.ds(bg * _L, _L)] = vals

                out_copy(i, h).start()
                return carry2

            lax.fori_loop(0, spans, span, 0)
            return carry

        lax.fori_loop(0, btiles_per_w, btile, 0)
        out_copy(btiles_per_w - 1, spans - 2).wait()
        out_copy(btiles_per_w - 1, spans - 1).wait()

    return sc_call


def kernel(amounts, emb):
    bsz, seq = amounts.shape
    out5 = _build_sc_call(bsz, seq)(amounts, emb)
    out = out5.transpose(2, 4, 0, 1, 3).reshape(bsz, seq, _EMB_DIM)
    return out
